# baseline (speedup 1.0000x reference)
import functools

import jax
import jax.numpy as jnp
from jax import lax
from jax.experimental import pallas as pl
from jax.experimental.pallas import tpu as pltpu
from jax.experimental.pallas import tpu_sc as plsc

B = 128
N = 512
F = 64
PE = 500
GE = 5000
HID = 128
GED = 128
CLIN = 6
NCLS = 2

BP = 8
NWORK = 32
PPW = B // NWORK
NWRD = N // 16
BMP = N * NWRD
PEP = 512
GEP = 5120


def _sc_build_body(pe_hbm, ges_hbm, ged_hbm, bm_hbm, outg_hbm,
                   zbuf, wbuf, ebuf, gsrc, gdst, ibuf, vbuf, sem):
    cid = lax.axis_index("c")
    sid = lax.axis_index("s")
    wid = sid * 2 + cid
    lane = lax.broadcasted_iota(jnp.int32, (16,), 0)

    @pl.when(jnp.logical_and(cid == 1, sid == 0))
    def _():
        def _fill_z(i, _):
            zbuf[pl.ds(i * 16, 16)] = jnp.zeros((16,), jnp.float32)
            return 0

        def _fill_o(i, _):
            vbuf[pl.ds(i * 16, 16)] = jnp.ones((16,), jnp.float32)
            return 0

        lax.fori_loop(0, B * B // 16, _fill_z, 0, unroll=8)
        lax.fori_loop(0, GEP // 16, _fill_o, 0, unroll=8)
        pltpu.sync_copy(zbuf, outg_hbm)
        pltpu.sync_copy(ges_hbm, gsrc)
        pltpu.sync_copy(ged_hbm, gdst)

        def _gidx(c, _):
            s = gsrc[pl.ds(c * 16, 16)]
            d = gdst[pl.ds(c * 16, 16)]
            ibuf[pl.ds(c * 16, 16)] = d * B + s
            return 0

        lax.fori_loop(0, GEP // 16, _gidx, 0, unroll=4)
        pltpu.async_copy(vbuf, outg_hbm.at[ibuf], sem).wait()

    for j in range(PPW):
        b = wid * PPW + j

        def _zero_w(i, _):
            wbuf[pl.ds(i * 16, 16)] = jnp.zeros((16,), jnp.int32)
            return 0

        lax.fori_loop(0, BMP // 16, _zero_w, 0, unroll=8)
        pltpu.sync_copy(pe_hbm.at[b], ebuf)

        def _edges(c, _):
            s = ebuf[0, pl.ds(c * 16, 16)]
            d = ebuf[1, pl.ds(c * 16, 16)]
            widx = d * NWRD + lax.shift_right_logical(s, 4)
            bit = lax.shift_left(jnp.full((16,), 1, jnp.int32),
                                 lax.bitwise_and(s, 15))
            mask = (c * 16 + lane) < PE
            plsc.addupdate_scatter(wbuf, [widx], bit, mask=mask)
            return 0

        lax.fori_loop(0, PEP // 16, _edges, 0, unroll=4)
        pltpu.sync_copy(wbuf, bm_hbm.at[pl.ds(b * BMP, BMP)])


@functools.partial(
    pl.kernel,
    out_type=(jax.ShapeDtypeStruct((B * BMP,), jnp.int32),
              jax.ShapeDtypeStruct((B * B,), jnp.float32)),
    mesh=plsc.VectorSubcoreMesh(core_axis_name="c", subcore_axis_name="s"),
    scratch_types=[
        pltpu.VMEM((B * B,), jnp.float32),
        pltpu.VMEM((BMP,), jnp.int32),
        pltpu.VMEM((2, PEP), jnp.int32),
        pltpu.VMEM((GEP,), jnp.int32),
        pltpu.VMEM((GEP,), jnp.int32),
        pltpu.VMEM((GEP,), jnp.int32),
        pltpu.VMEM((GEP,), jnp.float32),
        pltpu.SemaphoreType.DMA,
    ],
    compiler_params=pltpu.CompilerParams(needs_layout_passes=False),
)
def _sc_build(pe_hbm, ges_hbm, ged_hbm, bm_hbm, outg_hbm,
              zbuf, wbuf, ebuf, gsrc, gdst, ibuf, vbuf, sem):
    _sc_build_body(pe_hbm, ges_hbm, ged_hbm, bm_hbm, outg_hbm,
                   zbuf, wbuf, ebuf, gsrc, gdst, ibuf, vbuf, sem)


def _patient_body(a_ref, x_ref, w1_ref, b1_ref, w2_ref, b2_ref,
                  w3_ref, b3_ref, plw_ref, plb_ref, out_ref):
    w1 = w1_ref[...]
    w2 = w2_ref[...]
    w3 = w3_ref[...]
    b1 = b1_ref[...]
    b2 = b2_ref[...]
    b3 = b3_ref[...]
    sel = (lax.broadcasted_iota(jnp.int32, (NWRD, N), 1) // 16
           == lax.broadcasted_iota(jnp.int32, (NWRD, N), 0)
           ).astype(jnp.float32)
    shift = lax.broadcasted_iota(jnp.int32, (N, N), 1) & 15
    ones_col = jnp.full((N, 8), 1.0, jnp.float32)

    def _expand(p):
        bmf = a_ref[p].astype(jnp.float32)
        v = jnp.dot(bmf, sel, preferred_element_type=jnp.float32)
        return ((v.astype(jnp.int32) >> shift) & 1).astype(jnp.float32)

    nxt = _expand(0)
    for p in range(BP):
        abar = nxt
        if p + 1 < BP:
            nxt = _expand(p + 1)
        deg = jnp.dot(abar, ones_col,
                      preferred_element_type=jnp.float32)[:, 0:1] + 1.0
        dis = lax.rsqrt(deg)
        dis2 = dis * dis
        h = x_ref[p]
        for w, bb in ((w1, b1), (w2, b2), (w3, b3)):
            z = jnp.dot(h, w, preferred_element_type=jnp.float32)
            zn = dis * z
            acc = jnp.dot(abar, zn, preferred_element_type=jnp.float32)
            h = jnp.maximum(dis * acc + dis2 * z + bb, 0.0)
        g = jnp.max(h, axis=0, keepdims=True)
        out_ref[p:p + 1, :] = (
            jnp.dot(g, plw_ref[...], preferred_element_type=jnp.float32)
            + plb_ref[...])


def _group_body(ag_ref, emb_ref, demo_ref, w1a_ref, w1b_ref, b1_ref,
                w2_ref, b2_ref, w3_ref, b3_ref, w4_ref, b4_ref, out_ref):
    abar = ag_ref[...]
    deg = jnp.sum(abar, axis=1, keepdims=True) + 1.0
    dis = lax.rsqrt(deg)
    dis2 = dis * dis
    an = dis * abar * jnp.transpose(dis)

    z = (jnp.dot(emb_ref[...], w1a_ref[...], preferred_element_type=jnp.float32)
         + jnp.dot(demo_ref[...], w1b_ref[...], preferred_element_type=jnp.float32))
    h = jnp.maximum(jnp.dot(an, z, preferred_element_type=jnp.float32)
                    + dis2 * z + b1_ref[...], 0.0)
    for w_ref, b_ref, act in ((w2_ref, b2_ref, True), (w3_ref, b3_ref, True),
                              (w4_ref, b4_ref, False)):
        z = jnp.dot(h, w_ref[...], preferred_element_type=jnp.float32)
        h = (jnp.dot(an, z, preferred_element_type=jnp.float32)
             + dis2 * z + b_ref[...])
        if act:
            h = jnp.maximum(h, 0.0)
    m = jnp.max(h, axis=1, keepdims=True)
    y = h - m
    out_ref[...] = y - jnp.log(jnp.sum(jnp.exp(y), axis=1, keepdims=True))


def kernel(x, demographic, patient_edge_idx, group_edge_idx,
           pW1, pb1, pW2, pb2, pW3, pb3, plinW, plinb,
           gW1, gb1, gW2, gb2, gW3, gb3, gW4, gb4):
    pe_pad = jnp.concatenate(
        [patient_edge_idx,
         jnp.tile(patient_edge_idx[:, :, -1:], (1, 1, PEP - PE))], axis=2)
    ge_pad = jnp.concatenate(
        [group_edge_idx,
         jnp.tile(group_edge_idx[:, -1:], (1, GEP - GE))], axis=1)

    bm_flat, ag_flat = _sc_build(pe_pad, ge_pad[0], ge_pad[1])
    bm = bm_flat.reshape(B, N, NWRD)
    ag = ag_flat.reshape(B, B)

    row = lambda v: v.reshape(1, -1)
    fullg = lambda a: pl.BlockSpec(a.shape, lambda i: (0,) * a.ndim)
    full = lambda a: pl.BlockSpec(a.shape, lambda: (0,) * a.ndim)

    wargs = (pW1, row(pb1), pW2, row(pb2), pW3, row(pb3), plinW, row(plinb))
    embed = pl.pallas_call(
        _patient_body,
        grid=(B // BP,),
        in_specs=[
            pl.BlockSpec((BP, N, NWRD), lambda i: (i, 0, 0)),
            pl.BlockSpec((BP, N, F), lambda i: (i, 0, 0)),
        ] + [fullg(a) for a in wargs],
        out_specs=pl.BlockSpec((BP, GED), lambda i: (i, 0)),
        out_shape=jax.ShapeDtypeStruct((B, GED), jnp.float32),
    )(bm, x, *wargs)

    gw1a = gW1[:GED]
    gw1b = gW1[GED:]
    gargs = (ag, embed, demographic, gw1a, gw1b, row(gb1),
             gW2, row(gb2), gW3, row(gb3), gW4, row(gb4))
    out = pl.pallas_call(
        _group_body,
        in_specs=[full(a) for a in gargs],
        out_specs=pl.BlockSpec((B, NCLS), lambda: (0, 0)),
        out_shape=jax.ShapeDtypeStruct((B, NCLS), jnp.float32),
    )(*gargs)
    return out


# speedup vs baseline: 1.1129x; 1.1129x over previous
import functools

import jax
import jax.numpy as jnp
from jax import lax
from jax.experimental import pallas as pl
from jax.experimental.pallas import tpu as pltpu
from jax.experimental.pallas import tpu_sc as plsc

B = 128
N = 512
F = 64
PE = 500
GE = 5000
HID = 128
GED = 128
CLIN = 6
NCLS = 2

BP = 8
NWORK = 32
PPW = B // NWORK
NWRD = N // 16
BMP = N * NWRD
PEP = 512
GEP = 5120


def _sc_build_body(pe_hbm, ges_hbm, ged_hbm, bm_hbm, outg_hbm,
                   zbuf, wbuf, ebuf, gsrc, gdst, ibuf, vbuf, sem):
    cid = lax.axis_index("c")
    sid = lax.axis_index("s")
    wid = sid * 2 + cid
    lane = lax.broadcasted_iota(jnp.int32, (16,), 0)

    @pl.when(jnp.logical_and(cid == 1, sid == 0))
    def _():
        def _fill_z(i, _):
            zbuf[pl.ds(i * 16, 16)] = jnp.zeros((16,), jnp.float32)
            return 0

        def _fill_o(i, _):
            vbuf[pl.ds(i * 16, 16)] = jnp.ones((16,), jnp.float32)
            return 0

        lax.fori_loop(0, B * B // 16, _fill_z, 0, unroll=8)
        lax.fori_loop(0, GEP // 16, _fill_o, 0, unroll=8)
        pltpu.sync_copy(zbuf, outg_hbm)
        pltpu.sync_copy(ges_hbm, gsrc)
        pltpu.sync_copy(ged_hbm, gdst)

        def _gidx(c, _):
            s = gsrc[pl.ds(c * 16, 16)]
            d = gdst[pl.ds(c * 16, 16)]
            ibuf[pl.ds(c * 16, 16)] = d * B + s
            return 0

        lax.fori_loop(0, GEP // 16, _gidx, 0, unroll=4)
        pltpu.async_copy(vbuf, outg_hbm.at[ibuf], sem).wait()

    for j in range(PPW):
        b = wid * PPW + j

        def _zero_w(i, _):
            wbuf[pl.ds(i * 16, 16)] = jnp.zeros((16,), jnp.int32)
            return 0

        lax.fori_loop(0, BMP // 16, _zero_w, 0, unroll=8)
        pltpu.sync_copy(pe_hbm.at[b], ebuf)

        def _edges(c, _):
            s = ebuf[0, pl.ds(c * 16, 16)]
            d = ebuf[1, pl.ds(c * 16, 16)]
            widx = d * NWRD + lax.shift_right_logical(s, 4)
            bit = lax.shift_left(jnp.full((16,), 1, jnp.int32),
                                 lax.bitwise_and(s, 15))
            mask = (c * 16 + lane) < PE
            plsc.addupdate_scatter(wbuf, [widx], bit, mask=mask)
            return 0

        lax.fori_loop(0, PEP // 16, _edges, 0, unroll=4)
        pltpu.sync_copy(wbuf, bm_hbm.at[pl.ds(b * BMP, BMP)])


@functools.partial(
    pl.kernel,
    out_type=(jax.ShapeDtypeStruct((B * BMP,), jnp.int32),
              jax.ShapeDtypeStruct((B * B,), jnp.float32)),
    mesh=plsc.VectorSubcoreMesh(core_axis_name="c", subcore_axis_name="s"),
    scratch_types=[
        pltpu.VMEM((B * B,), jnp.float32),
        pltpu.VMEM((BMP,), jnp.int32),
        pltpu.VMEM((2, PEP), jnp.int32),
        pltpu.VMEM((GEP,), jnp.int32),
        pltpu.VMEM((GEP,), jnp.int32),
        pltpu.VMEM((GEP,), jnp.int32),
        pltpu.VMEM((GEP,), jnp.float32),
        pltpu.SemaphoreType.DMA,
    ],
    compiler_params=pltpu.CompilerParams(needs_layout_passes=False),
)
def _sc_build(pe_hbm, ges_hbm, ged_hbm, bm_hbm, outg_hbm,
              zbuf, wbuf, ebuf, gsrc, gdst, ibuf, vbuf, sem):
    _sc_build_body(pe_hbm, ges_hbm, ged_hbm, bm_hbm, outg_hbm,
                   zbuf, wbuf, ebuf, gsrc, gdst, ibuf, vbuf, sem)


def _patient_body(a_ref, x_ref, w1_ref, b1_ref, w2_ref, b2_ref,
                  w3_ref, b3_ref, plw_ref, plb_ref, out_ref):
    w1 = w1_ref[...]
    w2 = w2_ref[...]
    w3 = w3_ref[...]
    b1 = b1_ref[...]
    b2 = b2_ref[...]
    b3 = b3_ref[...]
    sel = (lax.broadcasted_iota(jnp.int32, (NWRD, N), 1) // 16
           == lax.broadcasted_iota(jnp.int32, (NWRD, N), 0)
           ).astype(jnp.float32)
    shift = lax.broadcasted_iota(jnp.int32, (N, N), 1) & 15
    ones_col = jnp.full((N, 8), 1.0, jnp.float32)

    def _expand(p):
        bmf = a_ref[p].astype(jnp.float32)
        v = jnp.dot(bmf, sel, preferred_element_type=jnp.float32)
        return ((v.astype(jnp.int32) >> shift) & 1).astype(jnp.float32)

    nxt = _expand(0)
    for p in range(BP):
        abar = nxt
        if p + 1 < BP:
            nxt = _expand(p + 1)
        deg = jnp.sum(abar, axis=1, keepdims=True) + 1.0
        dis = lax.rsqrt(deg)
        dis2 = dis * dis
        h = x_ref[p]
        for w, bb in ((w1, b1), (w2, b2), (w3, b3)):
            z = jnp.dot(h, w, preferred_element_type=jnp.float32)
            zn = dis * z
            acc = jnp.dot(abar, zn, preferred_element_type=jnp.float32)
            h = jnp.maximum(dis * acc + dis2 * z + bb, 0.0)
        g = jnp.max(h, axis=0, keepdims=True)
        out_ref[p:p + 1, :] = (
            jnp.dot(g, plw_ref[...], preferred_element_type=jnp.float32)
            + plb_ref[...])


def _group_body(ag_ref, emb_ref, demo_ref, w1a_ref, w1b_ref, b1_ref,
                w2_ref, b2_ref, w3_ref, b3_ref, w4_ref, b4_ref, out_ref):
    abar = ag_ref[...]
    deg = jnp.sum(abar, axis=1, keepdims=True) + 1.0
    dis = lax.rsqrt(deg)
    dis2 = dis * dis
    an = dis * abar * jnp.transpose(dis)

    z = (jnp.dot(emb_ref[...], w1a_ref[...], preferred_element_type=jnp.float32)
         + jnp.dot(demo_ref[...], w1b_ref[...], preferred_element_type=jnp.float32))
    h = jnp.maximum(jnp.dot(an, z, preferred_element_type=jnp.float32)
                    + dis2 * z + b1_ref[...], 0.0)
    for w_ref, b_ref, act in ((w2_ref, b2_ref, True), (w3_ref, b3_ref, True),
                              (w4_ref, b4_ref, False)):
        z = jnp.dot(h, w_ref[...], preferred_element_type=jnp.float32)
        h = (jnp.dot(an, z, preferred_element_type=jnp.float32)
             + dis2 * z + b_ref[...])
        if act:
            h = jnp.maximum(h, 0.0)
    m = jnp.max(h, axis=1, keepdims=True)
    y = h - m
    out_ref[...] = y - jnp.log(jnp.sum(jnp.exp(y), axis=1, keepdims=True))


def kernel(x, demographic, patient_edge_idx, group_edge_idx,
           pW1, pb1, pW2, pb2, pW3, pb3, plinW, plinb,
           gW1, gb1, gW2, gb2, gW3, gb3, gW4, gb4):
    pe_pad = jnp.concatenate(
        [patient_edge_idx,
         jnp.tile(patient_edge_idx[:, :, -1:], (1, 1, PEP - PE))], axis=2)
    ge_pad = jnp.concatenate(
        [group_edge_idx,
         jnp.tile(group_edge_idx[:, -1:], (1, GEP - GE))], axis=1)

    bm_flat, ag_flat = _sc_build(pe_pad, ge_pad[0], ge_pad[1])
    bm = bm_flat.reshape(B, N, NWRD)
    ag = ag_flat.reshape(B, B)

    row = lambda v: v.reshape(1, -1)
    fullg = lambda a: pl.BlockSpec(a.shape, lambda i: (0,) * a.ndim)
    full = lambda a: pl.BlockSpec(a.shape, lambda: (0,) * a.ndim)

    wargs = (pW1, row(pb1), pW2, row(pb2), pW3, row(pb3), plinW, row(plinb))
    embed = pl.pallas_call(
        _patient_body,
        grid=(B // BP,),
        in_specs=[
            pl.BlockSpec((BP, N, NWRD), lambda i: (i, 0, 0)),
            pl.BlockSpec((BP, N, F), lambda i: (i, 0, 0)),
        ] + [fullg(a) for a in wargs],
        out_specs=pl.BlockSpec((BP, GED), lambda i: (i, 0)),
        out_shape=jax.ShapeDtypeStruct((B, GED), jnp.float32),
    )(bm, x, *wargs)

    gw1a = gW1[:GED]
    gw1b = gW1[GED:]
    gargs = (ag, embed, demographic, gw1a, gw1b, row(gb1),
             gW2, row(gb2), gW3, row(gb3), gW4, row(gb4))
    out = pl.pallas_call(
        _group_body,
        in_specs=[full(a) for a in gargs],
        out_specs=pl.BlockSpec((B, NCLS), lambda: (0, 0)),
        out_shape=jax.ShapeDtypeStruct((B, NCLS), jnp.float32),
    )(*gargs)
    return out


# speedup vs baseline: 1.1134x; 1.0005x over previous
import functools

import jax
import jax.numpy as jnp
from jax import lax
from jax.experimental import pallas as pl
from jax.experimental.pallas import tpu as pltpu
from jax.experimental.pallas import tpu_sc as plsc

B = 128
N = 512
F = 64
PE = 500
GE = 5000
HID = 128
GED = 128
CLIN = 6
NCLS = 2

BP = 8
NWORK = 32
PPW = B // NWORK
NWRD = N // 16
BMP = N * NWRD
PEP = 512
GEP = 5120


def _sc_build_body(pe_hbm, ges_hbm, ged_hbm, bm_hbm, outg_hbm,
                   zbuf, wbuf, ebuf, gsrc, gdst, ibuf, vbuf, sem):
    cid = lax.axis_index("c")
    sid = lax.axis_index("s")
    wid = sid * 2 + cid
    lane = lax.broadcasted_iota(jnp.int32, (16,), 0)

    @pl.when(jnp.logical_and(cid == 1, sid == 0))
    def _():
        def _fill_z(i, _):
            zbuf[pl.ds(i * 16, 16)] = jnp.zeros((16,), jnp.float32)
            return 0

        def _fill_o(i, _):
            vbuf[pl.ds(i * 16, 16)] = jnp.ones((16,), jnp.float32)
            return 0

        lax.fori_loop(0, B * B // 16, _fill_z, 0, unroll=8)
        lax.fori_loop(0, GEP // 16, _fill_o, 0, unroll=8)
        pltpu.sync_copy(zbuf, outg_hbm)
        pltpu.sync_copy(ges_hbm, gsrc)
        pltpu.sync_copy(ged_hbm, gdst)

        def _gidx(c, _):
            s = gsrc[pl.ds(c * 16, 16)]
            d = gdst[pl.ds(c * 16, 16)]
            ibuf[pl.ds(c * 16, 16)] = d * B + s
            return 0

        lax.fori_loop(0, GEP // 16, _gidx, 0, unroll=4)
        pltpu.async_copy(vbuf, outg_hbm.at[ibuf], sem).wait()

    for j in range(PPW):
        b = wid * PPW + j

        def _zero_w(i, _):
            wbuf[pl.ds(i * 16, 16)] = jnp.zeros((16,), jnp.int32)
            return 0

        lax.fori_loop(0, BMP // 16, _zero_w, 0, unroll=8)
        pltpu.sync_copy(pe_hbm.at[b], ebuf)

        def _edges(c, _):
            s = ebuf[0, pl.ds(c * 16, 16)]
            d = ebuf[1, pl.ds(c * 16, 16)]
            widx = d * NWRD + lax.shift_right_logical(s, 4)
            bit = lax.shift_left(jnp.full((16,), 1, jnp.int32),
                                 lax.bitwise_and(s, 15))
            mask = (c * 16 + lane) < PE
            plsc.addupdate_scatter(wbuf, [widx], bit, mask=mask)
            return 0

        lax.fori_loop(0, PEP // 16, _edges, 0, unroll=4)
        pltpu.sync_copy(wbuf, bm_hbm.at[pl.ds(b * BMP, BMP)])


@functools.partial(
    pl.kernel,
    out_type=(jax.ShapeDtypeStruct((B * BMP,), jnp.int32),
              jax.ShapeDtypeStruct((B * B,), jnp.float32)),
    mesh=plsc.VectorSubcoreMesh(core_axis_name="c", subcore_axis_name="s"),
    scratch_types=[
        pltpu.VMEM((B * B,), jnp.float32),
        pltpu.VMEM((BMP,), jnp.int32),
        pltpu.VMEM((2, PEP), jnp.int32),
        pltpu.VMEM((GEP,), jnp.int32),
        pltpu.VMEM((GEP,), jnp.int32),
        pltpu.VMEM((GEP,), jnp.int32),
        pltpu.VMEM((GEP,), jnp.float32),
        pltpu.SemaphoreType.DMA,
    ],
    compiler_params=pltpu.CompilerParams(needs_layout_passes=False),
)
def _sc_build(pe_hbm, ges_hbm, ged_hbm, bm_hbm, outg_hbm,
              zbuf, wbuf, ebuf, gsrc, gdst, ibuf, vbuf, sem):
    _sc_build_body(pe_hbm, ges_hbm, ged_hbm, bm_hbm, outg_hbm,
                   zbuf, wbuf, ebuf, gsrc, gdst, ibuf, vbuf, sem)


def _patient_body(a_ref, x_ref, w1_ref, b1_ref, w2_ref, b2_ref,
                  w3_ref, b3_ref, plw_ref, plb_ref, out_ref):
    w1 = w1_ref[...]
    w2 = w2_ref[...]
    w3 = w3_ref[...]
    b1 = b1_ref[...]
    b2 = b2_ref[...]
    b3 = b3_ref[...]
    sel = (lax.broadcasted_iota(jnp.int32, (NWRD, N), 1) // 16
           == lax.broadcasted_iota(jnp.int32, (NWRD, N), 0)
           ).astype(jnp.float32)
    shift = lax.broadcasted_iota(jnp.int32, (N, N), 1) & 15

    def _expand(p):
        bmf = a_ref[p].astype(jnp.float32)
        v = jnp.dot(bmf, sel, preferred_element_type=jnp.float32)
        return ((v.astype(jnp.int32) >> shift) & 1).astype(jnp.float32)

    nxt = _expand(0)
    for p in range(BP):
        abar = nxt
        if p + 1 < BP:
            nxt = _expand(p + 1)
        deg = jnp.sum(abar, axis=1, keepdims=True) + 1.0
        dis = lax.rsqrt(deg)
        dis2 = dis * dis
        h = x_ref[p]
        for w, bb in ((w1, b1), (w2, b2), (w3, b3)):
            z = jnp.dot(h, w, preferred_element_type=jnp.float32)
            zn = dis * z
            acc = jnp.dot(abar, zn, preferred_element_type=jnp.float32)
            h = jnp.maximum(dis * acc + dis2 * z + bb, 0.0)
        g = jnp.max(h, axis=0, keepdims=True)
        out_ref[p:p + 1, :] = (
            jnp.dot(g, plw_ref[...], preferred_element_type=jnp.float32)
            + plb_ref[...])


def _group_body(ag_ref, emb_ref, demo_ref, w1a_ref, w1b_ref, b1_ref,
                w2_ref, b2_ref, w3_ref, b3_ref, w4_ref, b4_ref, out_ref):
    abar = ag_ref[...]
    deg = jnp.sum(abar, axis=1, keepdims=True) + 1.0
    dis = lax.rsqrt(deg)
    dis2 = dis * dis
    an = dis * abar * jnp.transpose(dis)

    z = (jnp.dot(emb_ref[...], w1a_ref[...], preferred_element_type=jnp.float32)
         + jnp.dot(demo_ref[...], w1b_ref[...], preferred_element_type=jnp.float32))
    h = jnp.maximum(jnp.dot(an, z, preferred_element_type=jnp.float32)
                    + dis2 * z + b1_ref[...], 0.0)
    for w_ref, b_ref, act in ((w2_ref, b2_ref, True), (w3_ref, b3_ref, True),
                              (w4_ref, b4_ref, False)):
        z = jnp.dot(h, w_ref[...], preferred_element_type=jnp.float32)
        h = (jnp.dot(an, z, preferred_element_type=jnp.float32)
             + dis2 * z + b_ref[...])
        if act:
            h = jnp.maximum(h, 0.0)
    m = jnp.max(h, axis=1, keepdims=True)
    y = h - m
    out_ref[...] = y - jnp.log(jnp.sum(jnp.exp(y), axis=1, keepdims=True))


def kernel(x, demographic, patient_edge_idx, group_edge_idx,
           pW1, pb1, pW2, pb2, pW3, pb3, plinW, plinb,
           gW1, gb1, gW2, gb2, gW3, gb3, gW4, gb4):
    pe_pad = jnp.concatenate(
        [patient_edge_idx,
         jnp.tile(patient_edge_idx[:, :, -1:], (1, 1, PEP - PE))], axis=2)
    ge_pad = jnp.concatenate(
        [group_edge_idx,
         jnp.tile(group_edge_idx[:, -1:], (1, GEP - GE))], axis=1)

    bm_flat, ag_flat = _sc_build(pe_pad, ge_pad[0], ge_pad[1])
    bm = bm_flat.reshape(B, N, NWRD)
    ag = ag_flat.reshape(B, B)

    row = lambda v: v.reshape(1, -1)
    fullg = lambda a: pl.BlockSpec(a.shape, lambda i: (0,) * a.ndim)
    full = lambda a: pl.BlockSpec(a.shape, lambda: (0,) * a.ndim)

    wargs = (pW1, row(pb1), pW2, row(pb2), pW3, row(pb3), plinW, row(plinb))
    embed = pl.pallas_call(
        _patient_body,
        grid=(B // BP,),
        in_specs=[
            pl.BlockSpec((BP, N, NWRD), lambda i: (i, 0, 0)),
            pl.BlockSpec((BP, N, F), lambda i: (i, 0, 0)),
        ] + [fullg(a) for a in wargs],
        out_specs=pl.BlockSpec((BP, GED), lambda i: (i, 0)),
        out_shape=jax.ShapeDtypeStruct((B, GED), jnp.float32),
    )(bm, x, *wargs)

    gw1a = gW1[:GED]
    gw1b = gW1[GED:]
    gargs = (ag, embed, demographic, gw1a, gw1b, row(gb1),
             gW2, row(gb2), gW3, row(gb3), gW4, row(gb4))
    out = pl.pallas_call(
        _group_body,
        in_specs=[full(a) for a in gargs],
        out_specs=pl.BlockSpec((B, NCLS), lambda: (0, 0)),
        out_shape=jax.ShapeDtypeStruct((B, NCLS), jnp.float32),
    )(*gargs)
    return out
